# baseline (speedup 1.0000x reference)
import functools

import jax
import jax.numpy as jnp
from jax import lax
from jax.experimental import pallas as pl
from jax.experimental.pallas import tpu as pltpu
from jax.experimental.pallas import tpu_sc as plsc

_K = 16
_NG = _K - 1
_EPS = 1e-3


def _dist_topk_cz_body(ng, xt_ref, xf_ref, wa_ref, wb_ref, cb_ref,
                       idx_ref, c_ref, z_ref):
    b = pl.program_id(0)
    xt = xt_ref[0]
    xfT = xf_ref[0]
    TN = xt.shape[0]
    N = xfT.shape[1]

    sqt = jnp.sum(xt * xt, axis=1, keepdims=True)
    sqf = jnp.sum(xfT * xfT, axis=0, keepdims=True)
    dot = lax.dot_general(xt, xfT, (((1,), (0,)), ((), ())),
                          preferred_element_type=jnp.float32)
    d = 2.0 * dot - sqt - sqf

    colf = lax.broadcasted_iota(jnp.int32, (TN, N), 1).astype(jnp.float32)
    rowf = (lax.broadcasted_iota(jnp.int32, (TN, 1), 0)
            + pl.program_id(1) * TN).astype(jnp.float32)
    cols = [] if ng == _NG else [rowf]
    dm = jnp.where(colf == rowf, -jnp.inf, d)
    for _ in range(_NG):
        m = jnp.max(dm, axis=1, keepdims=True)
        e = dm == m
        am = jnp.min(jnp.where(e, colf, jnp.float32(N)), axis=1,
                     keepdims=True)
        cols.append(am)
        dm = jnp.where(colf == am, -jnp.inf, dm)
    idx_ref[0] = (jnp.concatenate(cols, axis=1).astype(jnp.int32)
                  + b * N)

    D = wa_ref.shape[1]
    scale_z = cb_ref[0:1]
    scale = cb_ref[0:1, :D]
    beta = cb_ref[1:2, :D]
    c_ref[0] = jnp.dot(xt, wa_ref[...],
                       preferred_element_type=jnp.float32) * scale + beta
    z_ref[0] = jnp.dot(xt, wb_ref[...],
                       preferred_element_type=jnp.float32) * scale_z


def _block_tc(x, xT, wa, wb, scale_beta, tn, ng):
    B, N, C = x.shape
    D = wa.shape[1]
    DZ = wb.shape[1]
    grid = (B, N // tn)
    return pl.pallas_call(
        functools.partial(_dist_topk_cz_body, ng),
        grid=grid,
        in_specs=[
            pl.BlockSpec((1, tn, C), lambda b, n: (b, n, 0)),
            pl.BlockSpec((1, C, N), lambda b, n: (b, 0, 0)),
            pl.BlockSpec((C, D), lambda b, n: (0, 0)),
            pl.BlockSpec((C, DZ), lambda b, n: (0, 0)),
            pl.BlockSpec((2, DZ), lambda b, n: (0, 0)),
        ],
        out_specs=[
            pl.BlockSpec((1, tn, ng), lambda b, n: (b, n, 0)),
            pl.BlockSpec((1, tn, D), lambda b, n: (b, n, 0)),
            pl.BlockSpec((1, tn, DZ), lambda b, n: (b, n, 0)),
        ],
        out_shape=[
            jax.ShapeDtypeStruct((B, N, ng), jnp.int32),
            jax.ShapeDtypeStruct((B, N, D), jnp.float32),
            jax.ShapeDtypeStruct((B, N, DZ), jnp.float32),
        ],
    )(x, xT, wa, wb, scale_beta)


def _gathermax_sc(z, idx, c):
    M, DZ = z.shape
    D = c.shape[1]
    NG = idx.shape[0] // M
    NW = 32
    P = M // NW
    CP = 8 if NG == _NG else 4
    NCH = P // CP
    NB = 4
    mesh = plsc.VectorSubcoreMesh(core_axis_name="c", subcore_axis_name="s")

    @functools.partial(
        pl.kernel, mesh=mesh,
        out_type=jax.ShapeDtypeStruct((M, D), jnp.float32),
        scratch_types=[
            pltpu.VMEM((NB, CP * NG), jnp.int32),
            pltpu.VMEM((NB * CP * NG, DZ), jnp.float32),
            pltpu.VMEM((NB * CP, DZ), jnp.float32),
            pltpu.VMEM((NB * CP, D), jnp.float32),
            pltpu.VMEM((NB * CP, D), jnp.float32),
            pltpu.SemaphoreType.DMA((NB,)),
            pltpu.SemaphoreType.DMA((NB,)),
            pltpu.SemaphoreType.DMA((NB,)),
        ],
    )
    def k(z_hbm, idx_hbm, c_hbm, out_hbm, idx_v, rows_v, self_v, cin_v,
          cout_v, idx_sem, in_sem, out_sem):
        wid = lax.axis_index("s") * 2 + lax.axis_index("c")
        base = wid * P

        def idx_copy(ci, bb):
            return pltpu.make_async_copy(
                idx_hbm.at[pl.ds((base + ci * CP) * NG, CP * NG)],
                idx_v.at[bb], idx_sem.at[bb])

        def gather(ci, bb):
            return pltpu.make_async_copy(
                z_hbm.at[idx_v.at[bb]],
                rows_v.at[pl.ds(bb * CP * NG, CP * NG)], in_sem.at[bb])

        def self_copy(ci, bb):
            return pltpu.make_async_copy(
                z_hbm.at[pl.ds(base + ci * CP, CP)],
                self_v.at[pl.ds(bb * CP, CP)], in_sem.at[bb])

        def c_copy(ci, bb):
            return pltpu.make_async_copy(
                c_hbm.at[pl.ds(base + ci * CP, CP)],
                cin_v.at[pl.ds(bb * CP, CP)], in_sem.at[bb])

        def out_copy(ci, bb):
            return pltpu.make_async_copy(
                cout_v.at[pl.ds(bb * CP, CP)],
                out_hbm.at[pl.ds(base + ci * CP, CP)], out_sem.at[bb])

        def start_in(ci, bb):
            gather(ci, bb).start()
            if NG == _NG:
                self_copy(ci, bb).start()
            c_copy(ci, bb).start()

        def wait_in(ci, bb):
            gather(ci, bb).wait()
            if NG == _NG:
                self_copy(ci, bb).wait()
            c_copy(ci, bb).wait()

        for j in range(NB):
            idx_copy(j, j).start()
        for j in range(NB - 1):
            idx_copy(j, j).wait()
            start_in(j, j)

        def body(cj, carry):
            for bb in range(NB):
                ci = NB * cj + bb

                @pl.when(ci + NB - 1 < NCH)
                def _():
                    b3 = (bb + NB - 1) % NB
                    idx_copy(ci + NB - 1, b3).wait()
                    start_in(ci + NB - 1, b3)

                wait_in(ci, bb)

                @pl.when(ci + NB < NCH)
                def _():
                    idx_copy(ci + NB, bb).start()

                @pl.when(ci >= NB)
                def _():
                    out_copy(ci - NB, bb).wait()

                for p in range(CP):
                    r0 = bb * CP * NG + p * NG
                    for dc in range(D // 16):
                        sl = pl.ds(dc * 16, 16)
                        if NG == _NG:
                            leaves = [self_v[bb * CP + p, sl]]
                        else:
                            leaves = []
                        leaves += [rows_v[r0 + j, sl] for j in range(NG)]
                        while len(leaves) > 1:
                            leaves = [jnp.maximum(leaves[i], leaves[i + 1])
                                      for i in range(0, len(leaves) - 1, 2)] \
                                + ([leaves[-1]] if len(leaves) % 2 else [])
                        cout_v[bb * CP + p, sl] = jnp.maximum(
                            cin_v[bb * CP + p, sl] + leaves[0], 0.0)

                out_copy(ci, bb).start()
            return carry

        lax.fori_loop(0, NCH // NB, body, 0)
        for j in range(NB):
            out_copy(NCH - NB + j, (NCH - NB + j) % NB).wait()

    return k(z, idx, c)


def _final_body(x1_ref, x2_ref, x3_ref, x4_ref,
                w1_ref, w2_ref, w3_ref, w4_ref, cb_ref, o_ref):
    acc = jnp.dot(x1_ref[...], w1_ref[...], preferred_element_type=jnp.float32)
    acc = acc + jnp.dot(x2_ref[...], w2_ref[...],
                        preferred_element_type=jnp.float32)
    acc = acc + jnp.dot(x3_ref[...], w3_ref[...],
                        preferred_element_type=jnp.float32)
    acc = acc + jnp.dot(x4_ref[...], w4_ref[...],
                        preferred_element_type=jnp.float32)
    o_ref[...] = jnp.maximum(acc * cb_ref[0:1] + cb_ref[1:2], 0.0)


def _final_tc(x1, x2, x3, x4, w5, scale_beta, tm):
    M = x1.shape[0]
    Dout = w5.shape[1]
    d1, d2, d3 = x1.shape[1], x2.shape[1], x3.shape[1]
    d4 = x4.shape[1]
    w51 = w5[:d1]
    w52 = w5[d1:d1 + d2]
    w53 = w5[d1 + d2:d1 + d2 + d3]
    w54 = w5[d1 + d2 + d3:]
    grid = (M // tm,)
    return pl.pallas_call(
        _final_body,
        grid=grid,
        in_specs=[
            pl.BlockSpec((tm, d1), lambda i: (i, 0)),
            pl.BlockSpec((tm, d2), lambda i: (i, 0)),
            pl.BlockSpec((tm, d3), lambda i: (i, 0)),
            pl.BlockSpec((tm, d4), lambda i: (i, 0)),
            pl.BlockSpec((d1, Dout), lambda i: (0, 0)),
            pl.BlockSpec((d2, Dout), lambda i: (0, 0)),
            pl.BlockSpec((d3, Dout), lambda i: (0, 0)),
            pl.BlockSpec((d4, Dout), lambda i: (0, 0)),
            pl.BlockSpec((2, Dout), lambda i: (0, 0)),
        ],
        out_specs=pl.BlockSpec((tm, Dout), lambda i: (i, 0)),
        out_shape=jax.ShapeDtypeStruct((M, Dout), jnp.float32),
    )(x1, x2, x3, x4, w51, w52, w53, w54, scale_beta)


def _edge_block(x, w, gamma, beta, tn):
    B, N, C = x.shape
    D = w.shape[1]
    s = gamma / jnp.sqrt(jnp.float32(1.0) + _EPS)
    sb = jnp.stack([s, beta])
    wb = w[C:]
    if D % 128:
        dz = D + (-D) % 128
        wb = jnp.pad(wb, ((0, 0), (0, dz - D)))
        sb = jnp.pad(sb, ((0, 0), (0, dz - D)))
    NS = 4 if D >= 128 else 2
    ng = _NG if D % 128 else _K
    outs = []
    for h in range(NS):
        xh = x[h * (B // NS):(h + 1) * (B // NS)]
        xT = jnp.transpose(xh, (0, 2, 1))
        idx, c, z = _block_tc(xh, xT, w[:C], wb, sb, tn, ng)
        mh = (B // NS) * N
        outs.append(_gathermax_sc(z.reshape(mh, -1), idx.reshape(mh * ng),
                                  c.reshape(mh, D)).reshape(B // NS, N, D))
    return jnp.concatenate(outs, axis=0)


def kernel(x, W1, W2, W3, W4, W5, g1, b1, g2, b2, g3, b3, g4, b4, g5, b5):
    B, N, _ = x.shape
    x1 = _edge_block(x, W1, g1, b1, 1024)
    x2 = _edge_block(x1, W2, g2, b2, 1024)
    x3 = _edge_block(x2, W3, g3, b3, 1024)
    x4 = _edge_block(x3, W4, g4, b4, 1024)
    s5 = g5 / jnp.sqrt(jnp.float32(1.0) + _EPS)
    sb5 = jnp.stack([s5, b5])
    ys = []
    for q in range(4):
        sl = slice(q * (B // 4), (q + 1) * (B // 4))
        mq = (B // 4) * N
        ys.append(_final_tc(x1[sl].reshape(mq, -1), x2[sl].reshape(mq, -1),
                            x3[sl].reshape(mq, -1), x4[sl].reshape(mq, -1),
                            W5, sb5, 2048))
    return jnp.concatenate(ys, axis=0).reshape(B, N, -1)


# speedup vs baseline: 1.0628x; 1.0628x over previous
import functools

import jax
import jax.numpy as jnp
from jax import lax
from jax.experimental import pallas as pl
from jax.experimental.pallas import tpu as pltpu
from jax.experimental.pallas import tpu_sc as plsc

_K = 16
_NG = _K - 1
_EPS = 1e-3


def _dist_topk_cz_body(ng, xt_ref, xf_ref, wa_ref, wb_ref, cb_ref,
                       idx_ref, c_ref, z_ref):
    b = pl.program_id(0)
    xt = xt_ref[0]
    xfT = xf_ref[0]
    TN = xt.shape[0]
    N = xfT.shape[1]

    sqt = jnp.sum(xt * xt, axis=1, keepdims=True)
    sqf = jnp.sum(xfT * xfT, axis=0, keepdims=True)
    dot = lax.dot_general(xt, xfT, (((1,), (0,)), ((), ())),
                          preferred_element_type=jnp.float32)
    d = 2.0 * dot - sqt - sqf

    colf = lax.broadcasted_iota(jnp.int32, (TN, N), 1).astype(jnp.float32)
    rowf = (lax.broadcasted_iota(jnp.int32, (TN, 1), 0)
            + pl.program_id(1) * TN).astype(jnp.float32)
    cols = [] if ng == _NG else [rowf]
    dm = jnp.where(colf == rowf, -jnp.inf, d)
    for _ in range(_NG):
        m = jnp.max(dm, axis=1, keepdims=True)
        e = dm == m
        am = jnp.min(jnp.where(e, colf, jnp.float32(N)), axis=1,
                     keepdims=True)
        cols.append(am)
        dm = jnp.where(colf == am, -jnp.inf, dm)
    idx_ref[0] = (jnp.concatenate(cols, axis=1).astype(jnp.int32)
                  + b * N)

    D = wa_ref.shape[1]
    scale_z = cb_ref[0:1]
    scale = cb_ref[0:1, :D]
    beta = cb_ref[1:2, :D]
    c_ref[0] = jnp.dot(xt, wa_ref[...],
                       preferred_element_type=jnp.float32) * scale + beta
    z_ref[0] = jnp.dot(xt, wb_ref[...],
                       preferred_element_type=jnp.float32) * scale_z


def _block_tc(x, xT, wa, wb, scale_beta, tn, ng):
    B, N, C = x.shape
    D = wa.shape[1]
    DZ = wb.shape[1]
    grid = (B, N // tn)
    return pl.pallas_call(
        functools.partial(_dist_topk_cz_body, ng),
        grid=grid,
        in_specs=[
            pl.BlockSpec((1, tn, C), lambda b, n: (b, n, 0)),
            pl.BlockSpec((1, C, N), lambda b, n: (b, 0, 0)),
            pl.BlockSpec((C, D), lambda b, n: (0, 0)),
            pl.BlockSpec((C, DZ), lambda b, n: (0, 0)),
            pl.BlockSpec((2, DZ), lambda b, n: (0, 0)),
        ],
        out_specs=[
            pl.BlockSpec((1, tn, ng), lambda b, n: (b, n, 0)),
            pl.BlockSpec((1, tn, D), lambda b, n: (b, n, 0)),
            pl.BlockSpec((1, tn, DZ), lambda b, n: (b, n, 0)),
        ],
        out_shape=[
            jax.ShapeDtypeStruct((B, N, ng), jnp.int32),
            jax.ShapeDtypeStruct((B, N, D), jnp.float32),
            jax.ShapeDtypeStruct((B, N, DZ), jnp.float32),
        ],
    )(x, xT, wa, wb, scale_beta)


def _gathermax_sc(z, idx, c):
    M, DZ = z.shape
    D = c.shape[1]
    NG = idx.shape[0] // M
    NW = 32
    P = M // NW
    CP = 8 if NG == _NG else 4
    NCH = P // CP
    NB = 4
    mesh = plsc.VectorSubcoreMesh(core_axis_name="c", subcore_axis_name="s")

    @functools.partial(
        pl.kernel, mesh=mesh,
        out_type=jax.ShapeDtypeStruct((M, D), jnp.float32),
        scratch_types=[
            pltpu.VMEM((NB, CP * NG), jnp.int32),
            pltpu.VMEM((NB * CP * NG, DZ), jnp.float32),
            pltpu.VMEM((NB * CP, DZ), jnp.float32),
            pltpu.VMEM((NB * CP, D), jnp.float32),
            pltpu.VMEM((NB * CP, D), jnp.float32),
            pltpu.SemaphoreType.DMA((NB,)),
            pltpu.SemaphoreType.DMA((NB,)),
            pltpu.SemaphoreType.DMA((NB,)),
        ],
    )
    def k(z_hbm, idx_hbm, c_hbm, out_hbm, idx_v, rows_v, self_v, cin_v,
          cout_v, idx_sem, in_sem, out_sem):
        wid = lax.axis_index("s") * 2 + lax.axis_index("c")
        base = wid * P

        def idx_copy(ci, bb):
            return pltpu.make_async_copy(
                idx_hbm.at[pl.ds((base + ci * CP) * NG, CP * NG)],
                idx_v.at[bb], idx_sem.at[bb])

        def gather(ci, bb):
            return pltpu.make_async_copy(
                z_hbm.at[idx_v.at[bb]],
                rows_v.at[pl.ds(bb * CP * NG, CP * NG)], in_sem.at[bb])

        def self_copy(ci, bb):
            return pltpu.make_async_copy(
                z_hbm.at[pl.ds(base + ci * CP, CP)],
                self_v.at[pl.ds(bb * CP, CP)], in_sem.at[bb])

        def c_copy(ci, bb):
            return pltpu.make_async_copy(
                c_hbm.at[pl.ds(base + ci * CP, CP)],
                cin_v.at[pl.ds(bb * CP, CP)], in_sem.at[bb])

        def out_copy(ci, bb):
            return pltpu.make_async_copy(
                cout_v.at[pl.ds(bb * CP, CP)],
                out_hbm.at[pl.ds(base + ci * CP, CP)], out_sem.at[bb])

        def start_in(ci, bb):
            gather(ci, bb).start()
            if NG == _NG:
                self_copy(ci, bb).start()
            c_copy(ci, bb).start()

        def wait_in(ci, bb):
            gather(ci, bb).wait()
            if NG == _NG:
                self_copy(ci, bb).wait()
            c_copy(ci, bb).wait()

        for j in range(NB):
            idx_copy(j, j).start()
        for j in range(NB - 1):
            idx_copy(j, j).wait()
            start_in(j, j)

        def body(cj, carry):
            for bb in range(NB):
                ci = NB * cj + bb

                @pl.when(ci + NB - 1 < NCH)
                def _():
                    b3 = (bb + NB - 1) % NB
                    idx_copy(ci + NB - 1, b3).wait()
                    start_in(ci + NB - 1, b3)

                wait_in(ci, bb)

                @pl.when(ci + NB < NCH)
                def _():
                    idx_copy(ci + NB, bb).start()

                @pl.when(ci >= NB)
                def _():
                    out_copy(ci - NB, bb).wait()

                for p in range(CP):
                    r0 = bb * CP * NG + p * NG
                    for dc in range(D // 16):
                        sl = pl.ds(dc * 16, 16)
                        if NG == _NG:
                            leaves = [self_v[bb * CP + p, sl]]
                        else:
                            leaves = []
                        leaves += [rows_v[r0 + j, sl] for j in range(NG)]
                        while len(leaves) > 1:
                            leaves = [jnp.maximum(leaves[i], leaves[i + 1])
                                      for i in range(0, len(leaves) - 1, 2)] \
                                + ([leaves[-1]] if len(leaves) % 2 else [])
                        cout_v[bb * CP + p, sl] = jnp.maximum(
                            cin_v[bb * CP + p, sl] + leaves[0], 0.0)

                out_copy(ci, bb).start()
            return carry

        lax.fori_loop(0, NCH // NB, body, 0)
        for j in range(NB):
            out_copy(NCH - NB + j, (NCH - NB + j) % NB).wait()

    return k(z, idx, c)


def _final_body(x1_ref, x2_ref, x3_ref, x4_ref,
                w1_ref, w2_ref, w3_ref, w4_ref, cb_ref, o_ref):
    acc = jnp.dot(x1_ref[...], w1_ref[...], preferred_element_type=jnp.float32)
    acc = acc + jnp.dot(x2_ref[...], w2_ref[...],
                        preferred_element_type=jnp.float32)
    acc = acc + jnp.dot(x3_ref[...], w3_ref[...],
                        preferred_element_type=jnp.float32)
    acc = acc + jnp.dot(x4_ref[...], w4_ref[...],
                        preferred_element_type=jnp.float32)
    o_ref[...] = jnp.maximum(acc * cb_ref[0:1] + cb_ref[1:2], 0.0)


def _final_tc(x1, x2, x3, x4, w5, scale_beta, tm):
    M = x1.shape[0]
    Dout = w5.shape[1]
    d1, d2, d3 = x1.shape[1], x2.shape[1], x3.shape[1]
    d4 = x4.shape[1]
    w51 = w5[:d1]
    w52 = w5[d1:d1 + d2]
    w53 = w5[d1 + d2:d1 + d2 + d3]
    w54 = w5[d1 + d2 + d3:]
    grid = (M // tm,)
    return pl.pallas_call(
        _final_body,
        grid=grid,
        in_specs=[
            pl.BlockSpec((tm, d1), lambda i: (i, 0)),
            pl.BlockSpec((tm, d2), lambda i: (i, 0)),
            pl.BlockSpec((tm, d3), lambda i: (i, 0)),
            pl.BlockSpec((tm, d4), lambda i: (i, 0)),
            pl.BlockSpec((d1, Dout), lambda i: (0, 0)),
            pl.BlockSpec((d2, Dout), lambda i: (0, 0)),
            pl.BlockSpec((d3, Dout), lambda i: (0, 0)),
            pl.BlockSpec((d4, Dout), lambda i: (0, 0)),
            pl.BlockSpec((2, Dout), lambda i: (0, 0)),
        ],
        out_specs=pl.BlockSpec((tm, Dout), lambda i: (i, 0)),
        out_shape=jax.ShapeDtypeStruct((M, Dout), jnp.float32),
    )(x1, x2, x3, x4, w51, w52, w53, w54, scale_beta)


def _edge_block(x, w, gamma, beta, tn):
    B, N, C = x.shape
    D = w.shape[1]
    s = gamma / jnp.sqrt(jnp.float32(1.0) + _EPS)
    sb = jnp.stack([s, beta])
    wb = w[C:]
    if D % 128:
        dz = D + (-D) % 128
        wb = jnp.pad(wb, ((0, 0), (0, dz - D)))
        sb = jnp.pad(sb, ((0, 0), (0, dz - D)))
    NS = 4 if D >= 128 else 2
    ng = _NG if D % 128 else _K
    outs = []
    for h in range(NS):
        xh = x[h * (B // NS):(h + 1) * (B // NS)]
        xT = jnp.transpose(xh, (0, 2, 1))
        idx, c, z = _block_tc(xh, xT, w[:C], wb, sb, tn, ng)
        mh = (B // NS) * N
        outs.append(_gathermax_sc(z.reshape(mh, -1), idx.reshape(mh * ng),
                                  c.reshape(mh, D)).reshape(B // NS, N, D))
    return jnp.concatenate(outs, axis=0)


def kernel(x, W1, W2, W3, W4, W5, g1, b1, g2, b2, g3, b3, g4, b4, g5, b5):
    B, N, _ = x.shape
    x1 = _edge_block(x, W1, g1, b1, 512)
    x2 = _edge_block(x1, W2, g2, b2, 512)
    x3 = _edge_block(x2, W3, g3, b3, 512)
    x4 = _edge_block(x3, W4, g4, b4, 512)
    s5 = g5 / jnp.sqrt(jnp.float32(1.0) + _EPS)
    sb5 = jnp.stack([s5, b5])
    ys = []
    for q in range(4):
        sl = slice(q * (B // 4), (q + 1) * (B // 4))
        mq = (B // 4) * N
        ys.append(_final_tc(x1[sl].reshape(mq, -1), x2[sl].reshape(mq, -1),
                            x3[sl].reshape(mq, -1), x4[sl].reshape(mq, -1),
                            W5, sb5, 2048))
    return jnp.concatenate(ys, axis=0).reshape(B, N, -1)
